# baseline (speedup 1.0000x reference)
import functools

import jax
import jax.numpy as jnp
from jax.experimental import pallas as pl
from jax.experimental.pallas import tpu as pltpu

N = 10000
E = 160000
DIN = 128
DE = 16
HID = 256
L = 6
H = 8
C = HID // H
EH = HID // 4
G = 64

RB = 400
NRB = N // RB
EB = 2000
NEB = E // EB

_PREC = jax.lax.Precision.DEFAULT


def _ln(x, g, b):
    m = jnp.mean(x, axis=-1, keepdims=True)
    v = jnp.mean((x - m) * (x - m), axis=-1, keepdims=True)
    return (x - m) * jax.lax.rsqrt(v + 1e-5) * g + b


def _enc_body(x_ref, w_ref, b_ref, g_ref, be_ref, o_ref):
    y = jnp.dot(x_ref[...], w_ref[...], precision=_PREC,
                preferred_element_type=jnp.float32) + b_ref[...]
    o_ref[...] = jnp.maximum(_ln(y, g_ref[...], be_ref[...]), 0.0)


def _encoder(x, W, b, g, be):
    return pl.pallas_call(
        _enc_body,
        grid=(NRB,),
        in_specs=[
            pl.BlockSpec((RB, DIN), lambda i: (i, 0)),
            pl.BlockSpec((DIN, HID), lambda i: (0, 0)),
            pl.BlockSpec((1, HID), lambda i: (0, 0)),
            pl.BlockSpec((1, HID), lambda i: (0, 0)),
            pl.BlockSpec((1, HID), lambda i: (0, 0)),
        ],
        out_specs=pl.BlockSpec((RB, HID), lambda i: (i, 0)),
        out_shape=jax.ShapeDtypeStruct((N, HID), jnp.float32),
    )(x, W, b.reshape(1, HID), g.reshape(1, HID), be.reshape(1, HID))


def _lin_relu_body(x_ref, w_ref, b_ref, o_ref):
    y = jnp.dot(x_ref[...], w_ref[...], precision=_PREC,
                preferred_element_type=jnp.float32) + b_ref[...]
    o_ref[...] = jnp.maximum(y, 0.0)


def _edge_encoder(ea, W, b):
    return pl.pallas_call(
        _lin_relu_body,
        grid=(NEB,),
        in_specs=[
            pl.BlockSpec((EB, DE), lambda i: (i, 0)),
            pl.BlockSpec((DE, EH), lambda i: (0, 0)),
            pl.BlockSpec((1, EH), lambda i: (0, 0)),
        ],
        out_specs=pl.BlockSpec((EB, EH), lambda i: (i, 0)),
        out_shape=jax.ShapeDtypeStruct((E, EH), jnp.float32),
    )(ea, W, b.reshape(1, EH))


def _lin_body(x_ref, w_ref, b_ref, o_ref):
    o_ref[...] = jnp.dot(x_ref[...], w_ref[...], precision=_PREC,
                         preferred_element_type=jnp.float32) + b_ref[...]


def _linear(x, W, b, rb, din, dout):
    n = x.shape[0]
    return pl.pallas_call(
        _lin_body,
        grid=(n // rb,),
        in_specs=[
            pl.BlockSpec((rb, din), lambda i: (i, 0)),
            pl.BlockSpec((din, dout), lambda i: (0, 0)),
            pl.BlockSpec((1, dout), lambda i: (0, 0)),
        ],
        out_specs=pl.BlockSpec((rb, dout), lambda i: (i, 0)),
        out_shape=jax.ShapeDtypeStruct((n, dout), jnp.float32),
    )(x, W, b.reshape(1, dout))


def _post_body(agg_ref, hres_ref, bias_ref, n1g_ref, n1b_ref, f1W_ref,
               f1b_ref, f2W_ref, f2b_ref, n2g_ref, n2b_ref, o_ref):
    out = agg_ref[...] + bias_ref[...]
    h = jnp.maximum(_ln(out, n1g_ref[...], n1b_ref[...]), 0.0) + hres_ref[...]
    f = jnp.maximum(
        jnp.dot(h, f1W_ref[...], precision=_PREC,
                preferred_element_type=jnp.float32) + f1b_ref[...], 0.0)
    f = jnp.dot(f, f2W_ref[...], precision=_PREC,
                preferred_element_type=jnp.float32) + f2b_ref[...]
    o_ref[...] = _ln(f, n2g_ref[...], n2b_ref[...]) + h


def _post(agg, hres, d):
    row = lambda a: a.reshape(1, -1)
    return pl.pallas_call(
        _post_body,
        grid=(NRB,),
        in_specs=[
            pl.BlockSpec((RB, HID), lambda i: (i, 0)),
            pl.BlockSpec((RB, HID), lambda i: (i, 0)),
            pl.BlockSpec((1, HID), lambda i: (0, 0)),
            pl.BlockSpec((1, HID), lambda i: (0, 0)),
            pl.BlockSpec((1, HID), lambda i: (0, 0)),
            pl.BlockSpec((HID, 2 * HID), lambda i: (0, 0)),
            pl.BlockSpec((1, 2 * HID), lambda i: (0, 0)),
            pl.BlockSpec((2 * HID, HID), lambda i: (0, 0)),
            pl.BlockSpec((1, HID), lambda i: (0, 0)),
            pl.BlockSpec((1, HID), lambda i: (0, 0)),
            pl.BlockSpec((1, HID), lambda i: (0, 0)),
        ],
        out_specs=pl.BlockSpec((RB, HID), lambda i: (i, 0)),
        out_shape=jax.ShapeDtypeStruct((N, HID), jnp.float32),
    )(agg, hres, row(d['bias']), row(d['n1g']), row(d['n1b']), d['f1W'],
      row(d['f1b']), d['f2W'], row(d['f2b']), row(d['n2g']), row(d['n2b']))


def _pool_body(h_ref, b_ref, s_ref, c_ref, m_ref):
    i = pl.program_id(0)

    @pl.when(i == 0)
    def _init():
        s_ref[...] = jnp.zeros_like(s_ref)
        c_ref[...] = jnp.zeros_like(c_ref)
        m_ref[...] = jnp.full_like(m_ref, -jnp.inf)

    h = h_ref[...]
    seg = b_ref[0, 0, :]
    onehot = (seg[:, None] == jax.lax.iota(jnp.int32, G)[None, :])
    onef = onehot.astype(jnp.float32)
    s_ref[...] += jnp.dot(onef.T, h, precision=_PREC,
                          preferred_element_type=jnp.float32)
    c_ref[...] += jnp.sum(onef.T, axis=1, keepdims=True)

    def body(g, _):
        part = jnp.max(jnp.where(seg[:, None] == g, h, -jnp.inf), axis=0,
                       keepdims=True)
        m_ref[pl.ds(g, 1), :] = jnp.maximum(m_ref[pl.ds(g, 1), :], part)
        return 0

    jax.lax.fori_loop(0, G, body, 0)


def _pool(h, batch):
    batch3 = batch.astype(jnp.int32).reshape(NRB, 1, RB)
    return pl.pallas_call(
        _pool_body,
        grid=(NRB,),
        in_specs=[
            pl.BlockSpec((RB, HID), lambda i: (i, 0)),
            pl.BlockSpec((1, 1, RB), lambda i: (i, 0, 0)),
        ],
        out_specs=[
            pl.BlockSpec((G, HID), lambda i: (0, 0)),
            pl.BlockSpec((G, 1), lambda i: (0, 0)),
            pl.BlockSpec((G, HID), lambda i: (0, 0)),
        ],
        out_shape=[
            jax.ShapeDtypeStruct((G, HID), jnp.float32),
            jax.ShapeDtypeStruct((G, 1), jnp.float32),
            jax.ShapeDtypeStruct((G, HID), jnp.float32),
        ],
    )(h, batch3)


def _readout_body(s_ref, c_ref, m_ref, w1_ref, b1_ref, w2_ref, b2_ref,
                  w3_ref, b3_ref, o_ref):
    s = s_ref[...]
    cnt = c_ref[...]
    mean = s / jnp.maximum(cnt, 1.0)
    mx = jnp.where(cnt > 0, m_ref[...], 0.0)
    pool = jnp.concatenate([mean, mx, s], axis=1)
    r = jnp.maximum(jnp.dot(pool, w1_ref[...], precision=_PREC,
                            preferred_element_type=jnp.float32) + b1_ref[...], 0.0)
    r = jnp.maximum(jnp.dot(r, w2_ref[...], precision=_PREC,
                            preferred_element_type=jnp.float32) + b2_ref[...], 0.0)
    o_ref[...] = jnp.dot(r, w3_ref[...], precision=_PREC,
                         preferred_element_type=jnp.float32) + b3_ref[...]


def _readout(s, cnt, mx, p):
    row = lambda a: a.reshape(1, -1)
    return pl.pallas_call(
        _readout_body,
        out_shape=jax.ShapeDtypeStruct((G, 1), jnp.float32),
    )(s, cnt, mx, p['r1W'], row(p['r1b']), p['r2W'], row(p['r2b']),
      p['r3W'], row(p['r3b']))


def _edge_stage(xl, xr, ee, src, dst, att):
    xlh = xl.reshape(N, H, C)
    xrh = xr.reshape(N, H, C)
    eeh = ee.reshape(E, H, C)
    xj = xlh[src]
    t = jax.nn.leaky_relu(xj + xrh[dst] + eeh, 0.2)
    alpha = jnp.sum(t * att, axis=-1)
    m = jax.ops.segment_max(alpha, dst, N)
    m = jnp.where(jnp.isfinite(m), m, 0.0)
    ex = jnp.exp(alpha - m[dst])
    ssum = jax.ops.segment_sum(ex, dst, N)
    alpha = ex / (ssum[dst] + 1e-16)
    msg = xj * alpha[..., None]
    return jax.ops.segment_sum(msg, dst, N).reshape(N, HID)


def kernel(x, edge_index, edge_attr, batch, params):
    p = params
    src = edge_index[0]
    dst = edge_index[1]
    h = _encoder(x, p['enc_W'], p['enc_b'], p['enc_g'], p['enc_be'])
    e = _edge_encoder(edge_attr, p['ee_W'], p['ee_b'])
    for d in p['layers']:
        xlr = _linear(h, jnp.concatenate([d['Wl'], d['Wr']], axis=1),
                      jnp.concatenate([d['bl'], d['br']]), RB, HID, 2 * HID)
        xl = xlr[:, :HID]
        xr = xlr[:, HID:]
        ee = _linear(e, d['We'], d['be'], EB, EH, HID)
        agg = _edge_stage(xl, xr, ee, src, dst, d['att'])
        h = _post(agg, h, d)
    s, cnt, mx = _pool(h, batch)
    return _readout(s, cnt, mx, params)
